# Initial kernel scaffold; baseline (speedup 1.0000x reference)
import jax
import jax.numpy as jnp
from jax.experimental import pallas as pl


def kernel(c1_w, c1_b, c2_w, c2_b, fc1_w, fc1_b, fc2_w, fc2_b, x):
    raise NotImplementedError("write your pallas kernel here")


# speedup vs baseline: 58.5830x; 58.5830x over previous
import numpy as np
import jax
import jax.numpy as jnp
from jax.experimental import pallas as pl
from jax.experimental.pallas import tpu as pltpu

_BB = 256


def _sel(n_out, n_in, k, off):
    r = np.zeros((n_out, n_in, k), np.float32)
    p = np.arange(n_out)[:, None]
    t = np.arange(k)[None, :]
    r[p, 2 * p + off + t, t] = 1.0
    return r


_R1 = [_sel(12, 28, 5, d) for d in (0, 1)]
_R2 = [_sel(4, 12, 5, d) for d in (0, 1)]


def _net_kernel(x_ref, g0_ref, g1_ref, g2_ref, g3_ref,
                h0_ref, h1_ref, h2_ref, h3_ref,
                b1_ref, b2_ref, f1_ref, fb1_ref, f2_ref, fb2_ref, o_ref):
    x = x_ref[...].astype(jnp.bfloat16)

    z = jnp.dot(x, g0_ref[...], preferred_element_type=jnp.float32)
    z = jnp.maximum(z, jnp.dot(x, g1_ref[...], preferred_element_type=jnp.float32))
    z = jnp.maximum(z, jnp.dot(x, g2_ref[...], preferred_element_type=jnp.float32))
    z = jnp.maximum(z, jnp.dot(x, g3_ref[...], preferred_element_type=jnp.float32))
    a1 = jnp.maximum(z + b1_ref[...], 0.0).astype(jnp.bfloat16)

    z = jnp.dot(a1, h0_ref[...], preferred_element_type=jnp.float32)
    z = jnp.maximum(z, jnp.dot(a1, h1_ref[...], preferred_element_type=jnp.float32))
    z = jnp.maximum(z, jnp.dot(a1, h2_ref[...], preferred_element_type=jnp.float32))
    z = jnp.maximum(z, jnp.dot(a1, h3_ref[...], preferred_element_type=jnp.float32))
    a2 = jnp.maximum(z + b2_ref[...], 0.0).astype(jnp.bfloat16)

    h = jnp.dot(a2, f1_ref[...], preferred_element_type=jnp.float32) + fb1_ref[...]
    h = jnp.maximum(h, 0.0).astype(jnp.bfloat16)
    logits = jnp.dot(h, f2_ref[...], preferred_element_type=jnp.float32) + fb2_ref[...]
    m = jnp.max(logits, axis=-1, keepdims=True)
    lse = jnp.log(jnp.sum(jnp.exp(logits - m), axis=-1, keepdims=True)) + m
    o_ref[...] = logits - lse


def kernel(c1_w, c1_b, c2_w, c2_b, fc1_w, fc1_b, fc2_w, fc2_b, x):
    B = x.shape[0]
    xr = x.reshape(B, 28 * 28)

    w1 = c1_w[:, :10].reshape(5, 5, 10)
    w2 = c2_w[:, :20].reshape(5, 5, 10, 20)

    gs = [jnp.einsum('prh,qsw,hwc->rspqc', _R1[dh], _R1[dw], w1)
          .reshape(784, 1440).astype(jnp.bfloat16)
          for dh in (0, 1) for dw in (0, 1)]
    hs = [jnp.einsum('aph,bqw,hwic->pqiabc', _R2[dh], _R2[dw], w2)
          .reshape(1440, 320).astype(jnp.bfloat16)
          for dh in (0, 1) for dw in (0, 1)]

    b1l = jnp.tile(c1_b[0, :10], 144).reshape(1, 1440)
    b2l = jnp.tile(c2_b[0, :20], 16).reshape(1, 320)
    f1 = fc1_w.astype(jnp.bfloat16)
    f2 = fc2_w.astype(jnp.bfloat16)

    const = lambda shape: pl.BlockSpec(shape, lambda i: (0, 0))
    out = pl.pallas_call(
        _net_kernel,
        out_shape=jax.ShapeDtypeStruct((B, 128), jnp.float32),
        grid=(B // _BB,),
        in_specs=[pl.BlockSpec((_BB, 784), lambda i: (i, 0))]
                 + [const((784, 1440))] * 4
                 + [const((1440, 320))] * 4
                 + [const((1, 1440)), const((1, 320)),
                    const((320, 128)), const((1, 128)),
                    const((128, 128)), const((1, 128))],
        out_specs=pl.BlockSpec((_BB, 128), lambda i: (i, 0)),
        compiler_params=pltpu.CompilerParams(dimension_semantics=("parallel",)),
    )(xr, *gs, *hs, b1l, b2l, f1, fc1_b, f2, fc2_b)
    return out[:, :10]
